# Initial kernel scaffold; baseline (speedup 1.0000x reference)
import jax
import jax.numpy as jnp
from jax.experimental import pallas as pl


def kernel(x, real_weight, imag_weight):
    raise NotImplementedError("write your pallas kernel here")


# speedup vs baseline: 1.0442x; 1.0442x over previous
import functools

import jax
import jax.numpy as jnp
from jax import lax
from jax.experimental import pallas as pl
from jax.experimental.pallas import tpu as pltpu
from jax.experimental.pallas import tpu_sc as plsc

_VOCAB = 1000000
_D = 32
_B = 16384
_H = 50
_N = _B * _H
_NC = 2
_NS = 16
_NW = _NC * _NS
_RPW = _N // _NW
_CH = 128
_NCHUNK = _RPW // _CH
_NBUF = 4
_LA = 2

_mesh = plsc.VectorSubcoreMesh(
    core_axis_name="c", subcore_axis_name="s", num_cores=_NC, num_subcores=_NS
)


@functools.partial(
    pl.kernel,
    out_type=(
        jax.ShapeDtypeStruct((_N, _D), jnp.float32),
        jax.ShapeDtypeStruct((_N, _D), jnp.float32),
    ),
    mesh=_mesh,
    compiler_params=pltpu.CompilerParams(use_tc_tiling_on_sc=False),
    scratch_types=[
        pltpu.VMEM((_RPW,), jnp.int32),
        pltpu.VMEM((_NBUF, _CH, _D), jnp.float32),
        pltpu.VMEM((_NBUF, _CH, _D), jnp.float32),
        [pltpu.SemaphoreType.DMA] * _NBUF,
        [pltpu.SemaphoreType.DMA] * _NBUF,
    ],
)
def _sc_gather2(idx_hbm, rw_hbm, iw_hbm, rout_hbm, iout_hbm,
                idxall, rbuf, ibuf, sg, so):
    i32 = lambda v: jnp.asarray(v, jnp.int32)
    wid = i32(lax.axis_index("s")) * _NC + i32(lax.axis_index("c"))
    base = wid * _RPW

    pltpu.sync_copy(idx_hbm.at[pl.ds(base, _RPW)], idxall)

    def _idx_slice(t):
        return idxall.at[pl.ds(i32(t * _CH), _CH)]

    def _start(t, b):
        pltpu.async_copy(rw_hbm.at[_idx_slice(t)], rbuf.at[i32(b)], sg[b])
        pltpu.async_copy(iw_hbm.at[_idx_slice(t)], ibuf.at[i32(b)], sg[b])

    def _wait_gathers(t, b):
        pltpu.make_async_copy(rw_hbm.at[_idx_slice(t)], rbuf.at[i32(b)], sg[b]).wait()
        pltpu.make_async_copy(iw_hbm.at[_idx_slice(t)], ibuf.at[i32(b)], sg[b]).wait()

    def _out_slice(ref, t):
        return ref.at[pl.ds(base + i32(t) * _CH, _CH), :]

    def _wait_outs(t, b):
        pltpu.make_async_copy(rbuf.at[i32(b)], _out_slice(rout_hbm, t), so[b]).wait()
        pltpu.make_async_copy(ibuf.at[i32(b)], _out_slice(iout_hbm, t), so[b]).wait()

    for t0 in range(_LA):
        _start(t0, t0)

    def outer(tt, carry):
        for b in range(_NBUF):
            t = tt * _NBUF + b
            u = t + _LA
            ub = (b + _LA) % _NBUF

            @pl.when(u < _NCHUNK)
            def _():
                @pl.when(u >= _NBUF)
                def _():
                    _wait_outs(u - _NBUF, ub)
                _start(u, ub)

            _wait_gathers(t, b)
            pltpu.async_copy(rbuf.at[i32(b)], _out_slice(rout_hbm, t), so[b])
            pltpu.async_copy(ibuf.at[i32(b)], _out_slice(iout_hbm, t), so[b])
        return carry

    lax.fori_loop(jnp.int32(0), jnp.int32(_NCHUNK // _NBUF), outer, 0)

    for b in range(_NBUF):
        _wait_outs(_NCHUNK - _NBUF + b, (_NCHUNK - _NBUF + b) % _NBUF)


def kernel(x, real_weight, imag_weight):
    idx = x.reshape(_N).astype(jnp.int32)
    real, imag = _sc_gather2(idx, real_weight, imag_weight)
    out = lax.complex(real, imag)
    return out.reshape(_B, _H, _D)


# speedup vs baseline: 1.9087x; 1.8280x over previous
import functools

import jax
import jax.numpy as jnp
from jax import lax
from jax.experimental import pallas as pl
from jax.experimental.pallas import tpu as pltpu
from jax.experimental.pallas import tpu_sc as plsc

_VOCAB = 1000000
_D = 32
_B = 16384
_H = 50
_N = _B * _H
_NC = 2
_NS = 16
_NW = _NC * _NS
_BPW = _B // _NW
_CB = 128
_NSUB = _BPW // _CB
_NUNIT = _H * _NSUB

_mesh = plsc.VectorSubcoreMesh(
    core_axis_name="c", subcore_axis_name="s", num_cores=_NC, num_subcores=_NS
)


@functools.partial(
    pl.kernel,
    out_type=(
        jax.ShapeDtypeStruct((_H, _D, _B), jnp.float32),
        jax.ShapeDtypeStruct((_H, _D, _B), jnp.float32),
    ),
    mesh=_mesh,
    compiler_params=pltpu.CompilerParams(
        use_tc_tiling_on_sc=False, needs_layout_passes=False
    ),
    scratch_types=[
        pltpu.VMEM((_BPW * _H,), jnp.int32),
        pltpu.VMEM((2, _CB), jnp.int32),
        pltpu.VMEM((2, _CB, _D), jnp.float32),
        pltpu.VMEM((2, _CB, _D), jnp.float32),
        pltpu.VMEM((2, _D, _CB), jnp.float32),
        pltpu.VMEM((2, _D, _CB), jnp.float32),
        [pltpu.SemaphoreType.DMA] * 2,
        [pltpu.SemaphoreType.DMA] * 2,
    ],
)
def _sc_embed(idx_hbm, rw_hbm, iw_hbm, rout_hbm, iout_hbm,
              idxall, cvec, rbuf, ibuf, rt, it, sg, so):
    i32 = lambda v: jnp.asarray(v, jnp.int32)
    wid = i32(lax.axis_index("s")) * _NC + i32(lax.axis_index("c"))
    bbase = wid * _BPW
    ibase = bbase * _H

    pltpu.sync_copy(idx_hbm.at[pl.ds(ibase, _BPW * _H)], idxall)

    lanes = lax.iota(jnp.int32, 16)
    lanes_h = lanes * _H

    def _unit_hs(u):
        u = i32(u)
        return lax.shift_right_logical(u, jnp.int32(2)), u & jnp.int32(3)

    def _build_cvec(u, b):
        h, sub = _unit_hs(u)
        base = sub * (_CB * _H) + h
        for g in range(_CB // 16):
            pos = lanes_h + i32(base + g * (16 * _H))
            vals = plsc.load_gather(idxall, [pos])
            cvec[i32(b), pl.ds(g * 16, 16)] = vals

    def _start_gathers(b):
        pltpu.async_copy(rw_hbm.at[cvec.at[i32(b)]], rbuf.at[i32(b)], sg[b])
        pltpu.async_copy(iw_hbm.at[cvec.at[i32(b)]], ibuf.at[i32(b)], sg[b])

    def _wait_gathers(b):
        pltpu.make_async_copy(rw_hbm.at[cvec.at[i32(b)]], rbuf.at[i32(b)], sg[b]).wait()
        pltpu.make_async_copy(iw_hbm.at[cvec.at[i32(b)]], ibuf.at[i32(b)], sg[b]).wait()

    def _transpose(b):
        src_dst = ((rbuf, rt), (ibuf, it))
        for g in range(_CB // 16):
            rows = lanes + i32(g * 16)
            for src, dst in src_dst:
                s2 = src.at[i32(b)]
                for c in range(_D):
                    vals = plsc.load_gather(s2, [rows, jnp.full((16,), c, jnp.int32)])
                    dst[i32(b), c, pl.ds(g * 16, 16)] = vals

    def _out_slices(u):
        h, sub = _unit_hs(u)
        b0 = bbase + sub * _CB
        return (rout_hbm.at[h, :, pl.ds(b0, _CB)],
                iout_hbm.at[h, :, pl.ds(b0, _CB)])

    def _issue_outs(u, b):
        ro, io = _out_slices(u)
        pltpu.async_copy(rt.at[i32(b)], ro, so[b])
        pltpu.async_copy(it.at[i32(b)], io, so[b])

    def _wait_outs(u, b):
        ro, io = _out_slices(u)
        pltpu.make_async_copy(rt.at[i32(b)], ro, so[b]).wait()
        pltpu.make_async_copy(it.at[i32(b)], io, so[b]).wait()

    _build_cvec(i32(0), 0)
    _start_gathers(0)

    def outer(tt, carry):
        for b in range(2):
            u = tt * 2 + b
            nb = 1 - b
            _wait_gathers(b)

            @pl.when(u + 1 < _NUNIT)
            def _():
                _build_cvec(u + 1, nb)
                _start_gathers(nb)

            @pl.when(u >= 2)
            def _():
                _wait_outs(u - 2, b)

            _transpose(b)
            _issue_outs(u, b)
        return carry

    lax.fori_loop(jnp.int32(0), jnp.int32(_NUNIT // 2), outer, 0)

    for b in range(2):
        _wait_outs(_NUNIT - 2 + b, b)


def kernel(x, real_weight, imag_weight):
    idx = x.reshape(_N).astype(jnp.int32)
    rp, ip = _sc_embed(idx, real_weight, imag_weight)
    r = jnp.transpose(rp, (2, 0, 1))
    i = jnp.transpose(ip, (2, 0, 1))
    return lax.complex(r, i)


# speedup vs baseline: 2.2329x; 1.1699x over previous
import functools

import jax
import jax.numpy as jnp
from jax import lax
from jax.experimental import pallas as pl
from jax.experimental.pallas import tpu as pltpu
from jax.experimental.pallas import tpu_sc as plsc

_VOCAB = 1000000
_D = 32
_B = 16384
_H = 50
_N = _B * _H
_NC = 2
_NS = 16
_NW = _NC * _NS
_BPW = _B // _NW
_CB = 128
_NSUB = _BPW // _CB
_NUNIT = _H * _NSUB

_mesh = plsc.VectorSubcoreMesh(
    core_axis_name="c", subcore_axis_name="s", num_cores=_NC, num_subcores=_NS
)


@functools.partial(
    pl.kernel,
    out_type=(
        jax.ShapeDtypeStruct((_H, _D, _B), jnp.float32),
        jax.ShapeDtypeStruct((_H, _D, _B), jnp.float32),
    ),
    mesh=_mesh,
    compiler_params=pltpu.CompilerParams(
        use_tc_tiling_on_sc=False, needs_layout_passes=False
    ),
    scratch_types=[
        pltpu.VMEM((_BPW * _H,), jnp.int32),
        [pltpu.VMEM((_CB,), jnp.int32)] * 2,
        [pltpu.VMEM((_CB, _D), jnp.float32)] * 2,
        [pltpu.VMEM((_CB, _D), jnp.float32)] * 2,
        [pltpu.VMEM((_D, _CB), jnp.float32)] * 2,
        [pltpu.VMEM((_D, _CB), jnp.float32)] * 2,
        [pltpu.SemaphoreType.DMA] * 2,
        [pltpu.SemaphoreType.DMA] * 2,
    ],
)
def _sc_embed(idx_hbm, rw_hbm, iw_hbm, rout_hbm, iout_hbm,
              idxall, cvec, rbuf, ibuf, rt, it, sg, so):
    i32 = lambda v: jnp.asarray(v, jnp.int32)
    wid = i32(lax.axis_index("s")) * _NC + i32(lax.axis_index("c"))
    bbase = wid * _BPW
    ibase = bbase * _H

    pltpu.sync_copy(idx_hbm.at[pl.ds(ibase, _BPW * _H)], idxall)

    lanes = lax.iota(jnp.int32, 16)
    lanes_h = lanes * _H

    def _unit_hs(u):
        u = i32(u)
        return lax.shift_right_logical(u, jnp.int32(2)), u & jnp.int32(3)

    def _build_cvec(u, b):
        h, sub = _unit_hs(u)
        base = sub * (_CB * _H) + h
        cv = cvec[b]

        @plsc.parallel_loop(jnp.int32(0), jnp.int32(_CB // 16), jnp.int32(1), unroll=4)
        def _(g):
            pos = lanes_h + (base + g * (16 * _H))
            cv[pl.ds(g * 16, 16)] = plsc.load_gather(idxall, [pos])

    def _start_gathers(b):
        pltpu.async_copy(rw_hbm.at[cvec[b]], rbuf[b], sg[b])
        pltpu.async_copy(iw_hbm.at[cvec[b]], ibuf[b], sg[b])

    def _wait_gathers(b):
        pltpu.make_async_copy(rw_hbm.at[cvec[b]], rbuf[b], sg[b]).wait()
        pltpu.make_async_copy(iw_hbm.at[cvec[b]], ibuf[b], sg[b]).wait()

    def _transpose(b):
        rb, ib, rtb, itb = rbuf[b], ibuf[b], rt[b], it[b]

        @plsc.parallel_loop(jnp.int32(0), jnp.int32(_D * (_CB // 16)),
                            jnp.int32(1), unroll=8)
        def _(k):
            g = lax.shift_right_logical(k, jnp.int32(5))
            c = k & jnp.int32(31)
            rows = lanes + g * 16
            cols = jnp.zeros((16,), jnp.int32) + c
            rtb[c, pl.ds(g * 16, 16)] = plsc.load_gather(rb, [rows, cols])
            itb[c, pl.ds(g * 16, 16)] = plsc.load_gather(ib, [rows, cols])

    def _out_slices(u):
        h, sub = _unit_hs(u)
        b0 = bbase + sub * _CB
        return (rout_hbm.at[h, :, pl.ds(b0, _CB)],
                iout_hbm.at[h, :, pl.ds(b0, _CB)])

    def _issue_outs(u, b):
        ro, io = _out_slices(u)
        pltpu.async_copy(rt[b], ro, so[b])
        pltpu.async_copy(it[b], io, so[b])

    def _wait_outs(u, b):
        ro, io = _out_slices(u)
        pltpu.make_async_copy(rt[b], ro, so[b]).wait()
        pltpu.make_async_copy(it[b], io, so[b]).wait()

    _build_cvec(i32(0), 0)
    _start_gathers(0)

    def outer(tt, carry):
        for b in range(2):
            u = tt * 2 + b
            nb = 1 - b
            _wait_gathers(b)

            @pl.when(u + 1 < _NUNIT)
            def _():
                _build_cvec(u + 1, nb)
                _start_gathers(nb)

            @pl.when(u >= 2)
            def _():
                _wait_outs(u - 2, b)

            _transpose(b)
            _issue_outs(u, b)
        return carry

    lax.fori_loop(jnp.int32(0), jnp.int32(_NUNIT // 2), outer, 0)

    for b in range(2):
        _wait_outs(_NUNIT - 2 + b, b)


def kernel(x, real_weight, imag_weight):
    idx = x.reshape(_N).astype(jnp.int32)
    rp, ip = _sc_embed(idx, real_weight, imag_weight)
    r = jnp.transpose(rp, (2, 0, 1))
    i = jnp.transpose(ip, (2, 0, 1))
    return lax.complex(r, i)


# speedup vs baseline: 2.2447x; 1.0053x over previous
import functools

import jax
import jax.numpy as jnp
from jax import lax
from jax.experimental import pallas as pl
from jax.experimental.pallas import tpu as pltpu
from jax.experimental.pallas import tpu_sc as plsc

_VOCAB = 1000000
_D = 32
_B = 16384
_H = 50
_N = _B * _H
_NC = 2
_NS = 16
_NW = _NC * _NS
_BPW = _B // _NW
_CB = 128
_NSUB = _BPW // _CB
_NUNIT = _H * _NSUB

_mesh = plsc.VectorSubcoreMesh(
    core_axis_name="c", subcore_axis_name="s", num_cores=_NC, num_subcores=_NS
)

_VC = 128
_NFULL = _VOCAB // _VC
_VTAIL = _VOCAB - _NFULL * _VC
_CPW = (_NFULL + _NW - 1) // _NW


@functools.partial(
    pl.kernel,
    out_type=(
        jax.ShapeDtypeStruct((_VOCAB * _D,), jnp.float32),
        jax.ShapeDtypeStruct((_VOCAB * _D,), jnp.float32),
    ),
    mesh=_mesh,
    compiler_params=pltpu.CompilerParams(
        use_tc_tiling_on_sc=True, needs_layout_passes=False
    ),
    scratch_types=[
        [pltpu.VMEM((4, 8, _VC), jnp.float32)] * 2,
        [pltpu.VMEM((4, 8, _VC), jnp.float32)] * 2,
        [pltpu.VMEM((_VC * _D,), jnp.float32)] * 2,
        [pltpu.VMEM((_VC * _D,), jnp.float32)] * 2,
        [pltpu.SemaphoreType.DMA] * 2,
        [pltpu.SemaphoreType.DMA] * 2,
    ],
)
def _sc_repack(rwt_hbm, iwt_hbm, rtl_hbm, itl_hbm, rpk_hbm, ipk_hbm,
               vr, vi, tr, ti, sg, so):
    i32 = lambda v: jnp.asarray(v, jnp.int32)
    wid = i32(lax.axis_index("s")) * _NC + i32(lax.axis_index("c"))
    lanes = lax.iota(jnp.int32, 16)
    lanes_d = lanes * _D

    def _chunk_of(u):
        return i32(u) * _NW + wid

    def _starts(j, b):
        v0 = i32(j) * _VC
        for t in range(4):
            pltpu.async_copy(rwt_hbm.at[pl.ds(i32(t * 8), 8), pl.ds(v0, _VC)],
                             vr[b].at[i32(t)], sg[b])
            pltpu.async_copy(iwt_hbm.at[pl.ds(i32(t * 8), 8), pl.ds(v0, _VC)],
                             vi[b].at[i32(t)], sg[b])

    def _wait_in(j, b):
        v0 = i32(j) * _VC
        for t in range(4):
            pltpu.make_async_copy(rwt_hbm.at[pl.ds(i32(t * 8), 8), pl.ds(v0, _VC)],
                                  vr[b].at[i32(t)], sg[b]).wait()
            pltpu.make_async_copy(iwt_hbm.at[pl.ds(i32(t * 8), 8), pl.ds(v0, _VC)],
                                  vi[b].at[i32(t)], sg[b]).wait()

    def _transpose(b):
        vrb, vib, trb, tib = vr[b], vi[b], tr[b], ti[b]

        @plsc.parallel_loop(jnp.int32(0), jnp.int32(_D * (_VC // 16)),
                            jnp.int32(1), unroll=8)
        def _(k):
            g = lax.shift_right_logical(k, jnp.int32(5))
            c = k & jnp.int32(31)
            t = lax.shift_right_logical(c, jnp.int32(3))
            s = c & jnp.int32(7)
            pos = lanes_d + (g * (16 * _D) + c)
            plsc.store_scatter(trb, [pos], vrb[t, s, pl.ds(g * 16, 16)])
            plsc.store_scatter(tib, [pos], vib[t, s, pl.ds(g * 16, 16)])

    def _out_slices(j):
        o0 = i32(j) * (_VC * _D)
        return (rpk_hbm.at[pl.ds(o0, _VC * _D)],
                ipk_hbm.at[pl.ds(o0, _VC * _D)])

    def _issue_outs(j, b):
        ro, io = _out_slices(j)
        pltpu.async_copy(tr[b], ro, so[b])
        pltpu.async_copy(ti[b], io, so[b])

    def _wait_outs(j, b):
        ro, io = _out_slices(j)
        pltpu.make_async_copy(tr[b], ro, so[b]).wait()
        pltpu.make_async_copy(ti[b], io, so[b]).wait()

    _starts(_chunk_of(0), 0)

    def outer(tt, carry):
        for b in range(2):
            u = tt * 2 + b
            j = _chunk_of(u)
            nb = 1 - b

            @pl.when(j < _NFULL)
            def _():
                _wait_in(j, b)

                nj = _chunk_of(u + 1)

                @pl.when(nj < _NFULL)
                def _():
                    _starts(nj, nb)

                @pl.when(u >= 2)
                def _():
                    _wait_outs(_chunk_of(u - 2), b)

                _transpose(b)
                _issue_outs(j, b)
        return carry

    lax.fori_loop(jnp.int32(0), jnp.int32((_CPW + 1) // 2), outer, 0)

    for b in range(2):
        _wait_outs(_chunk_of(0), b)

    @pl.when(wid == _NW - 1)
    def _():
        o0 = _NFULL * _VC * _D
        n = _VTAIL * _D
        pltpu.sync_copy(rtl_hbm, tr[0].at[pl.ds(i32(0), n)])
        pltpu.sync_copy(tr[0].at[pl.ds(i32(0), n)], rpk_hbm.at[pl.ds(i32(o0), n)])
        pltpu.sync_copy(itl_hbm, ti[0].at[pl.ds(i32(0), n)])
        pltpu.sync_copy(ti[0].at[pl.ds(i32(0), n)], ipk_hbm.at[pl.ds(i32(o0), n)])


@functools.partial(
    pl.kernel,
    out_type=(
        jax.ShapeDtypeStruct((_H, _D, _B), jnp.float32),
        jax.ShapeDtypeStruct((_H, _D, _B), jnp.float32),
    ),
    mesh=_mesh,
    compiler_params=pltpu.CompilerParams(
        use_tc_tiling_on_sc=False, needs_layout_passes=False
    ),
    scratch_types=[
        pltpu.VMEM((_BPW * _H,), jnp.int32),
        [pltpu.VMEM((_CB,), jnp.int32)] * 2,
        [pltpu.VMEM((_CB, _D), jnp.float32)] * 2,
        [pltpu.VMEM((_CB, _D), jnp.float32)] * 2,
        [pltpu.VMEM((_D, _CB), jnp.float32)] * 2,
        [pltpu.VMEM((_D, _CB), jnp.float32)] * 2,
        [pltpu.SemaphoreType.DMA] * 2,
        [pltpu.SemaphoreType.DMA] * 2,
    ],
)
def _sc_embed(idx_hbm, rw_hbm, iw_hbm, rout_hbm, iout_hbm,
              idxall, cvec, rbuf, ibuf, rt, it, sg, so):
    i32 = lambda v: jnp.asarray(v, jnp.int32)
    wid = i32(lax.axis_index("s")) * _NC + i32(lax.axis_index("c"))
    bbase = wid * _BPW
    ibase = bbase * _H

    pltpu.sync_copy(idx_hbm.at[pl.ds(ibase, _BPW * _H)], idxall)

    lanes = lax.iota(jnp.int32, 16)
    lanes_h = lanes * _H

    def _unit_hs(u):
        u = i32(u)
        return lax.shift_right_logical(u, jnp.int32(2)), u & jnp.int32(3)

    def _build_cvec(u, b):
        h, sub = _unit_hs(u)
        base = sub * (_CB * _H) + h
        cv = cvec[b]

        @plsc.parallel_loop(jnp.int32(0), jnp.int32(_CB // 16), jnp.int32(1),
                            unroll=4)
        def _(g):
            pos = lanes_h + (base + g * (16 * _H))
            cv[pl.ds(g * 16, 16)] = plsc.load_gather(idxall, [pos])

    def _start_gathers(b):
        pltpu.async_copy(rw_hbm.at[cvec[b]], rbuf[b], sg[b])
        pltpu.async_copy(iw_hbm.at[cvec[b]], ibuf[b], sg[b])

    def _wait_gathers(b):
        pltpu.make_async_copy(rw_hbm.at[cvec[b]], rbuf[b], sg[b]).wait()
        pltpu.make_async_copy(iw_hbm.at[cvec[b]], ibuf[b], sg[b]).wait()

    def _transpose(b):
        rb, ib, rtb, itb = rbuf[b], ibuf[b], rt[b], it[b]

        @plsc.parallel_loop(jnp.int32(0), jnp.int32(_D * (_CB // 16)),
                            jnp.int32(1), unroll=8)
        def _(k):
            g = lax.shift_right_logical(k, jnp.int32(5))
            c = k & jnp.int32(31)
            rows = lanes + g * 16
            cols = jnp.zeros((16,), jnp.int32) + c
            rtb[c, pl.ds(g * 16, 16)] = plsc.load_gather(rb, [rows, cols])
            itb[c, pl.ds(g * 16, 16)] = plsc.load_gather(ib, [rows, cols])

    def _out_slices(u):
        h, sub = _unit_hs(u)
        b0 = bbase + sub * _CB
        return (rout_hbm.at[h, :, pl.ds(b0, _CB)],
                iout_hbm.at[h, :, pl.ds(b0, _CB)])

    def _issue_outs(u, b):
        ro, io = _out_slices(u)
        pltpu.async_copy(rt[b], ro, so[b])
        pltpu.async_copy(it[b], io, so[b])

    def _wait_outs(u, b):
        ro, io = _out_slices(u)
        pltpu.make_async_copy(rt[b], ro, so[b]).wait()
        pltpu.make_async_copy(it[b], io, so[b]).wait()

    _build_cvec(i32(0), 0)
    _start_gathers(0)

    def outer(tt, carry):
        for b in range(2):
            u = tt * 2 + b
            nb = 1 - b
            _wait_gathers(b)

            @pl.when(u + 1 < _NUNIT)
            def _():
                _build_cvec(u + 1, nb)
                _start_gathers(nb)

            @pl.when(u >= 2)
            def _():
                _wait_outs(u - 2, b)

            _transpose(b)
            _issue_outs(u, b)
        return carry

    lax.fori_loop(jnp.int32(0), jnp.int32(_NUNIT // 2), outer, 0)

    for b in range(2):
        _wait_outs(_NUNIT - 2 + b, b)


def kernel(x, real_weight, imag_weight):
    idx = x.reshape(_N).astype(jnp.int32)
    rtl = jnp.reshape(real_weight[_NFULL * _VC:], (_VTAIL * _D,))
    itl = jnp.reshape(imag_weight[_NFULL * _VC:], (_VTAIL * _D,))
    rpk, ipk = _sc_repack(real_weight.T, imag_weight.T, rtl, itl)
    rtab = jnp.reshape(rpk, (_VOCAB, _D))
    itab = jnp.reshape(ipk, (_VOCAB, _D))
    rp, ip = _sc_embed(idx, rtab, itab)
    r = jnp.transpose(rp, (2, 0, 1))
    i = jnp.transpose(ip, (2, 0, 1))
    return lax.complex(r, i)


# speedup vs baseline: 2.6201x; 1.1672x over previous
import functools

import jax
import jax.numpy as jnp
from jax import lax
from jax.experimental import pallas as pl
from jax.experimental.pallas import tpu as pltpu
from jax.experimental.pallas import tpu_sc as plsc

_VOCAB = 1000000
_D = 32
_B = 16384
_H = 50
_N = _B * _H
_NC = 2
_NS = 16
_NW = _NC * _NS
_BPW = _B // _NW
_CB = 128
_CBP = 133
_NSUB = _BPW // _CB
_NUNIT = _H * _NSUB

_mesh = plsc.VectorSubcoreMesh(
    core_axis_name="c", subcore_axis_name="s", num_cores=_NC, num_subcores=_NS
)

_VC = 128
_VCP = 133
_NFULL = _VOCAB // _VC
_VTAIL = _VOCAB - _NFULL * _VC
_CPW = (_NFULL + _NW - 1) // _NW


@functools.partial(
    pl.kernel,
    out_type=(
        jax.ShapeDtypeStruct((_VOCAB * _D,), jnp.float32),
        jax.ShapeDtypeStruct((_VOCAB * _D,), jnp.float32),
    ),
    mesh=_mesh,
    compiler_params=pltpu.CompilerParams(
        use_tc_tiling_on_sc=True, needs_layout_passes=False
    ),
    scratch_types=[
        [pltpu.VMEM((4, 8, _VCP), jnp.float32)] * 2,
        [pltpu.VMEM((4, 8, _VCP), jnp.float32)] * 2,
        [pltpu.VMEM((_VC * _D,), jnp.float32)] * 2,
        [pltpu.VMEM((_VC * _D,), jnp.float32)] * 2,
        [pltpu.SemaphoreType.DMA] * 2,
        [pltpu.SemaphoreType.DMA] * 2,
    ],
)
def _sc_repack(rwt_hbm, iwt_hbm, rtl_hbm, itl_hbm, rpk_hbm, ipk_hbm,
               vr, vi, tr, ti, sg, so):
    i32 = lambda v: jnp.asarray(v, jnp.int32)
    wid = i32(lax.axis_index("s")) * _NC + i32(lax.axis_index("c"))
    lanes = lax.iota(jnp.int32, 16)

    def _chunk_of(u):
        return i32(u) * _NW + wid

    def _starts(j, b):
        v0 = i32(j) * _VC
        for t in range(4):
            pltpu.async_copy(rwt_hbm.at[pl.ds(i32(t * 8), 8), pl.ds(v0, _VC)],
                             vr[b].at[i32(t), :, pl.ds(i32(0), _VC)], sg[b])
            pltpu.async_copy(iwt_hbm.at[pl.ds(i32(t * 8), 8), pl.ds(v0, _VC)],
                             vi[b].at[i32(t), :, pl.ds(i32(0), _VC)], sg[b])

    def _wait_in(j, b):
        v0 = i32(j) * _VC
        for t in range(4):
            pltpu.make_async_copy(rwt_hbm.at[pl.ds(i32(t * 8), 8), pl.ds(v0, _VC)],
                                  vr[b].at[i32(t), :, pl.ds(i32(0), _VC)], sg[b]).wait()
            pltpu.make_async_copy(iwt_hbm.at[pl.ds(i32(t * 8), 8), pl.ds(v0, _VC)],
                                  vi[b].at[i32(t), :, pl.ds(i32(0), _VC)], sg[b]).wait()

    c0 = lanes
    c1 = lanes + 16
    tv0 = lax.shift_right_logical(c0, jnp.int32(3))
    sv0 = c0 & jnp.int32(7)
    tv1 = lax.shift_right_logical(c1, jnp.int32(3))
    sv1 = c1 & jnp.int32(7)

    def _transpose(b):
        vrb, vib, trb, tib = vr[b], vi[b], tr[b], ti[b]

        @plsc.parallel_loop(jnp.int32(0), jnp.int32(_VC), jnp.int32(1),
                            unroll=8)
        def _(v):
            lv = jnp.zeros((16,), jnp.int32) + v
            o = v * _D
            trb[pl.ds(o, 16)] = plsc.load_gather(vrb, [tv0, sv0, lv])
            trb[pl.ds(o + 16, 16)] = plsc.load_gather(vrb, [tv1, sv1, lv])
            tib[pl.ds(o, 16)] = plsc.load_gather(vib, [tv0, sv0, lv])
            tib[pl.ds(o + 16, 16)] = plsc.load_gather(vib, [tv1, sv1, lv])

    def _out_slices(j):
        o0 = i32(j) * (_VC * _D)
        return (rpk_hbm.at[pl.ds(o0, _VC * _D)],
                ipk_hbm.at[pl.ds(o0, _VC * _D)])

    def _issue_outs(j, b):
        ro, io = _out_slices(j)
        pltpu.async_copy(tr[b], ro, so[b])
        pltpu.async_copy(ti[b], io, so[b])

    def _wait_outs(j, b):
        ro, io = _out_slices(j)
        pltpu.make_async_copy(tr[b], ro, so[b]).wait()
        pltpu.make_async_copy(ti[b], io, so[b]).wait()

    _starts(_chunk_of(0), 0)

    def outer(tt, carry):
        for b in range(2):
            u = tt * 2 + b
            j = _chunk_of(u)
            nb = 1 - b

            @pl.when(j < _NFULL)
            def _():
                _wait_in(j, b)

                nj = _chunk_of(u + 1)

                @pl.when(nj < _NFULL)
                def _():
                    _starts(nj, nb)

                @pl.when(u >= 2)
                def _():
                    _wait_outs(_chunk_of(u - 2), b)

                _transpose(b)
                _issue_outs(j, b)
        return carry

    lax.fori_loop(jnp.int32(0), jnp.int32((_CPW + 1) // 2), outer, 0)

    for b in range(2):
        _wait_outs(_chunk_of(0), b)

    @pl.when(wid == _NW - 1)
    def _():
        o0 = _NFULL * _VC * _D
        n = _VTAIL * _D
        pltpu.sync_copy(rtl_hbm, tr[0].at[pl.ds(i32(0), n)])
        pltpu.sync_copy(tr[0].at[pl.ds(i32(0), n)], rpk_hbm.at[pl.ds(i32(o0), n)])
        pltpu.sync_copy(itl_hbm, ti[0].at[pl.ds(i32(0), n)])
        pltpu.sync_copy(ti[0].at[pl.ds(i32(0), n)], ipk_hbm.at[pl.ds(i32(o0), n)])


@functools.partial(
    pl.kernel,
    out_type=(
        jax.ShapeDtypeStruct((_H, _D, _B), jnp.float32),
        jax.ShapeDtypeStruct((_H, _D, _B), jnp.float32),
    ),
    mesh=_mesh,
    compiler_params=pltpu.CompilerParams(
        use_tc_tiling_on_sc=False, needs_layout_passes=False
    ),
    scratch_types=[
        pltpu.VMEM((_BPW * _H,), jnp.int32),
        [pltpu.VMEM((_CB,), jnp.int32)] * 2,
        [pltpu.VMEM((_CB, _D), jnp.float32)] * 2,
        [pltpu.VMEM((_CB, _D), jnp.float32)] * 2,
        [pltpu.VMEM((_D, _CBP), jnp.float32)] * 2,
        [pltpu.VMEM((_D, _CBP), jnp.float32)] * 2,
        [pltpu.SemaphoreType.DMA] * 2,
        [pltpu.SemaphoreType.DMA] * 2,
    ],
)
def _sc_embed(idx_hbm, rw_hbm, iw_hbm, rout_hbm, iout_hbm,
              idxall, cvec, rbuf, ibuf, rt, it, sg, so):
    i32 = lambda v: jnp.asarray(v, jnp.int32)
    wid = i32(lax.axis_index("s")) * _NC + i32(lax.axis_index("c"))
    bbase = wid * _BPW
    ibase = bbase * _H

    pltpu.sync_copy(idx_hbm.at[pl.ds(ibase, _BPW * _H)], idxall)

    lanes = lax.iota(jnp.int32, 16)
    lanes_h = lanes * _H

    def _unit_hs(u):
        u = i32(u)
        return lax.shift_right_logical(u, jnp.int32(2)), u & jnp.int32(3)

    def _build_cvec(u, b):
        h, sub = _unit_hs(u)
        base = sub * (_CB * _H) + h
        cv = cvec[b]

        @plsc.parallel_loop(jnp.int32(0), jnp.int32(_CB // 16), jnp.int32(1),
                            unroll=4)
        def _(g):
            pos = lanes_h + (base + g * (16 * _H))
            cv[pl.ds(g * 16, 16)] = plsc.load_gather(idxall, [pos])

    def _start_gathers(b):
        pltpu.async_copy(rw_hbm.at[cvec[b]], rbuf[b], sg[b])
        pltpu.async_copy(iw_hbm.at[cvec[b]], ibuf[b], sg[b])

    def _wait_gathers(b):
        pltpu.make_async_copy(rw_hbm.at[cvec[b]], rbuf[b], sg[b]).wait()
        pltpu.make_async_copy(iw_hbm.at[cvec[b]], ibuf[b], sg[b]).wait()

    comp0 = lanes
    comp1 = lanes + 16

    def _transpose(b):
        rb, ib, rtb, itb = rbuf[b], ibuf[b], rt[b], it[b]

        @plsc.parallel_loop(jnp.int32(0), jnp.int32(_CB), jnp.int32(1),
                            unroll=8)
        def _(r):
            bs = jnp.zeros((16,), jnp.int32) + r
            plsc.store_scatter(rtb, [comp0, bs], rb[r, pl.ds(0, 16)])
            plsc.store_scatter(rtb, [comp1, bs], rb[r, pl.ds(16, 16)])
            plsc.store_scatter(itb, [comp0, bs], ib[r, pl.ds(0, 16)])
            plsc.store_scatter(itb, [comp1, bs], ib[r, pl.ds(16, 16)])

    def _out_slices(u):
        h, sub = _unit_hs(u)
        b0 = bbase + sub * _CB
        return (rout_hbm.at[h, :, pl.ds(b0, _CB)],
                iout_hbm.at[h, :, pl.ds(b0, _CB)])

    def _issue_outs(u, b):
        ro, io = _out_slices(u)
        pltpu.async_copy(rt[b].at[:, pl.ds(i32(0), _CB)], ro, so[b])
        pltpu.async_copy(it[b].at[:, pl.ds(i32(0), _CB)], io, so[b])

    def _wait_outs(u, b):
        ro, io = _out_slices(u)
        pltpu.make_async_copy(rt[b].at[:, pl.ds(i32(0), _CB)], ro, so[b]).wait()
        pltpu.make_async_copy(it[b].at[:, pl.ds(i32(0), _CB)], io, so[b]).wait()

    _build_cvec(i32(0), 0)
    _start_gathers(0)

    def outer(tt, carry):
        for b in range(2):
            u = tt * 2 + b
            nb = 1 - b
            _wait_gathers(b)

            @pl.when(u + 1 < _NUNIT)
            def _():
                _build_cvec(u + 1, nb)
                _start_gathers(nb)

            @pl.when(u >= 2)
            def _():
                _wait_outs(u - 2, b)

            _transpose(b)
            _issue_outs(u, b)
        return carry

    lax.fori_loop(jnp.int32(0), jnp.int32(_NUNIT // 2), outer, 0)

    for b in range(2):
        _wait_outs(_NUNIT - 2 + b, b)


def kernel(x, real_weight, imag_weight):
    idx = x.reshape(_N).astype(jnp.int32)
    rtl = jnp.reshape(real_weight[_NFULL * _VC:], (_VTAIL * _D,))
    itl = jnp.reshape(imag_weight[_NFULL * _VC:], (_VTAIL * _D,))
    rpk, ipk = _sc_repack(real_weight.T, imag_weight.T, rtl, itl)
    rtab = jnp.reshape(rpk, (_VOCAB, _D))
    itab = jnp.reshape(ipk, (_VOCAB, _D))
    rp, ip = _sc_embed(idx, rtab, itab)
    r = jnp.transpose(rp, (2, 0, 1))
    i = jnp.transpose(ip, (2, 0, 1))
    return lax.complex(r, i)


# speedup vs baseline: 2.6245x; 1.0017x over previous
import functools

import jax
import jax.numpy as jnp
from jax import lax
from jax.experimental import pallas as pl
from jax.experimental.pallas import tpu as pltpu
from jax.experimental.pallas import tpu_sc as plsc

_VOCAB = 1000000
_D = 32
_B = 16384
_H = 50
_N = _B * _H
_NC = 2
_NS = 16
_NW = _NC * _NS
_BPW = _B // _NW
_CB = 128
_CBP = 133
_NSUB = _BPW // _CB
_NUNIT = _H * _NSUB

_mesh = plsc.VectorSubcoreMesh(
    core_axis_name="c", subcore_axis_name="s", num_cores=_NC, num_subcores=_NS
)

_VC = 128
_VCP = 133
_NFULL = _VOCAB // _VC
_VTAIL = _VOCAB - _NFULL * _VC
_CPW = (_NFULL + _NW - 1) // _NW


@functools.partial(
    pl.kernel,
    out_type=(
        jax.ShapeDtypeStruct((_VOCAB * _D,), jnp.float32),
        jax.ShapeDtypeStruct((_VOCAB * _D,), jnp.float32),
    ),
    mesh=_mesh,
    compiler_params=pltpu.CompilerParams(
        use_tc_tiling_on_sc=True, needs_layout_passes=False
    ),
    scratch_types=[
        [pltpu.VMEM((_D, _VCP), jnp.float32)] * 2,
        [pltpu.VMEM((_D, _VCP), jnp.float32)] * 2,
        [pltpu.VMEM((_VC * _D,), jnp.float32)] * 2,
        [pltpu.VMEM((_VC * _D,), jnp.float32)] * 2,
        [pltpu.SemaphoreType.DMA] * 2,
        [pltpu.SemaphoreType.DMA] * 2,
    ],
)
def _sc_repack(rwt_hbm, iwt_hbm, rtl_hbm, itl_hbm, rpk_hbm, ipk_hbm,
               vr, vi, tr, ti, sg, so):
    i32 = lambda v: jnp.asarray(v, jnp.int32)
    wid = i32(lax.axis_index("s")) * _NC + i32(lax.axis_index("c"))
    lanes = lax.iota(jnp.int32, 16)

    def _chunk_of(u):
        return i32(u) * _NW + wid

    def _starts(j, b):
        v0 = i32(j) * _VC
        pltpu.async_copy(rwt_hbm.at[:, pl.ds(v0, _VC)],
                         vr[b].at[:, pl.ds(i32(0), _VC)], sg[b])
        pltpu.async_copy(iwt_hbm.at[:, pl.ds(v0, _VC)],
                         vi[b].at[:, pl.ds(i32(0), _VC)], sg[b])

    def _wait_in(j, b):
        v0 = i32(j) * _VC
        pltpu.make_async_copy(rwt_hbm.at[:, pl.ds(v0, _VC)],
                              vr[b].at[:, pl.ds(i32(0), _VC)], sg[b]).wait()
        pltpu.make_async_copy(iwt_hbm.at[:, pl.ds(v0, _VC)],
                              vi[b].at[:, pl.ds(i32(0), _VC)], sg[b]).wait()

    c0 = lanes
    c1 = lanes + 16

    def _transpose(b):
        vrb, vib, trb, tib = vr[b], vi[b], tr[b], ti[b]

        @plsc.parallel_loop(jnp.int32(0), jnp.int32(_VC), jnp.int32(1),
                            unroll=8)
        def _(v):
            lv = jnp.zeros((16,), jnp.int32) + v
            o = v * _D
            trb[pl.ds(o, 16)] = plsc.load_gather(vrb, [c0, lv])
            trb[pl.ds(o + 16, 16)] = plsc.load_gather(vrb, [c1, lv])
            tib[pl.ds(o, 16)] = plsc.load_gather(vib, [c0, lv])
            tib[pl.ds(o + 16, 16)] = plsc.load_gather(vib, [c1, lv])

    def _out_slices(j):
        o0 = i32(j) * (_VC * _D)
        return (rpk_hbm.at[pl.ds(o0, _VC * _D)],
                ipk_hbm.at[pl.ds(o0, _VC * _D)])

    def _issue_outs(j, b):
        ro, io = _out_slices(j)
        pltpu.async_copy(tr[b], ro, so[b])
        pltpu.async_copy(ti[b], io, so[b])

    def _wait_outs(j, b):
        ro, io = _out_slices(j)
        pltpu.make_async_copy(tr[b], ro, so[b]).wait()
        pltpu.make_async_copy(ti[b], io, so[b]).wait()

    _starts(_chunk_of(0), 0)

    def outer(tt, carry):
        for b in range(2):
            u = tt * 2 + b
            j = _chunk_of(u)
            nb = 1 - b

            @pl.when(j < _NFULL)
            def _():
                _wait_in(j, b)

                nj = _chunk_of(u + 1)

                @pl.when(nj < _NFULL)
                def _():
                    _starts(nj, nb)

                @pl.when(u >= 2)
                def _():
                    _wait_outs(_chunk_of(u - 2), b)

                _transpose(b)
                _issue_outs(j, b)
        return carry

    lax.fori_loop(jnp.int32(0), jnp.int32((_CPW + 1) // 2), outer, 0)

    for b in range(2):
        _wait_outs(_chunk_of(0), b)

    @pl.when(wid == _NW - 1)
    def _():
        o0 = _NFULL * _VC * _D
        n = _VTAIL * _D
        pltpu.sync_copy(rtl_hbm, tr[0].at[pl.ds(i32(0), n)])
        pltpu.sync_copy(tr[0].at[pl.ds(i32(0), n)], rpk_hbm.at[pl.ds(i32(o0), n)])
        pltpu.sync_copy(itl_hbm, ti[0].at[pl.ds(i32(0), n)])
        pltpu.sync_copy(ti[0].at[pl.ds(i32(0), n)], ipk_hbm.at[pl.ds(i32(o0), n)])


@functools.partial(
    pl.kernel,
    out_type=(
        jax.ShapeDtypeStruct((_H, _D, _B), jnp.float32),
        jax.ShapeDtypeStruct((_H, _D, _B), jnp.float32),
    ),
    mesh=_mesh,
    compiler_params=pltpu.CompilerParams(
        use_tc_tiling_on_sc=False, needs_layout_passes=False
    ),
    scratch_types=[
        pltpu.VMEM((_BPW * _H,), jnp.int32),
        [pltpu.VMEM((_CB,), jnp.int32)] * 2,
        [pltpu.VMEM((_CB, _D), jnp.float32)] * 2,
        [pltpu.VMEM((_CB, _D), jnp.float32)] * 2,
        [pltpu.VMEM((_D, _CBP), jnp.float32)] * 2,
        [pltpu.VMEM((_D, _CBP), jnp.float32)] * 2,
        [pltpu.SemaphoreType.DMA] * 2,
        [pltpu.SemaphoreType.DMA] * 2,
    ],
)
def _sc_embed(idx_hbm, rw_hbm, iw_hbm, rout_hbm, iout_hbm,
              idxall, cvec, rbuf, ibuf, rt, it, sg, so):
    i32 = lambda v: jnp.asarray(v, jnp.int32)
    wid = i32(lax.axis_index("s")) * _NC + i32(lax.axis_index("c"))
    bbase = wid * _BPW
    ibase = bbase * _H

    pltpu.sync_copy(idx_hbm.at[pl.ds(ibase, _BPW * _H)], idxall)

    lanes = lax.iota(jnp.int32, 16)
    lanes_h = lanes * _H

    def _unit_hs(u):
        u = i32(u)
        return lax.shift_right_logical(u, jnp.int32(2)), u & jnp.int32(3)

    def _build_cvec(u, b):
        h, sub = _unit_hs(u)
        base = sub * (_CB * _H) + h
        cv = cvec[b]

        @plsc.parallel_loop(jnp.int32(0), jnp.int32(_CB // 16), jnp.int32(1),
                            unroll=4)
        def _(g):
            pos = lanes_h + (base + g * (16 * _H))
            cv[pl.ds(g * 16, 16)] = plsc.load_gather(idxall, [pos])

    def _start_gathers(b):
        pltpu.async_copy(rw_hbm.at[cvec[b]], rbuf[b], sg[b])
        pltpu.async_copy(iw_hbm.at[cvec[b]], ibuf[b], sg[b])

    def _wait_gathers(b):
        pltpu.make_async_copy(rw_hbm.at[cvec[b]], rbuf[b], sg[b]).wait()
        pltpu.make_async_copy(iw_hbm.at[cvec[b]], ibuf[b], sg[b]).wait()

    comp0 = lanes
    comp1 = lanes + 16

    def _transpose(b):
        rb, ib, rtb, itb = rbuf[b], ibuf[b], rt[b], it[b]

        @plsc.parallel_loop(jnp.int32(0), jnp.int32(_CB), jnp.int32(1),
                            unroll=8)
        def _(r):
            bs = jnp.zeros((16,), jnp.int32) + r
            plsc.store_scatter(rtb, [comp0, bs], rb[r, pl.ds(0, 16)])
            plsc.store_scatter(rtb, [comp1, bs], rb[r, pl.ds(16, 16)])
            plsc.store_scatter(itb, [comp0, bs], ib[r, pl.ds(0, 16)])
            plsc.store_scatter(itb, [comp1, bs], ib[r, pl.ds(16, 16)])

    def _out_slices(u):
        h, sub = _unit_hs(u)
        b0 = bbase + sub * _CB
        return (rout_hbm.at[h, :, pl.ds(b0, _CB)],
                iout_hbm.at[h, :, pl.ds(b0, _CB)])

    def _issue_outs(u, b):
        ro, io = _out_slices(u)
        pltpu.async_copy(rt[b].at[:, pl.ds(i32(0), _CB)], ro, so[b])
        pltpu.async_copy(it[b].at[:, pl.ds(i32(0), _CB)], io, so[b])

    def _wait_outs(u, b):
        ro, io = _out_slices(u)
        pltpu.make_async_copy(rt[b].at[:, pl.ds(i32(0), _CB)], ro, so[b]).wait()
        pltpu.make_async_copy(it[b].at[:, pl.ds(i32(0), _CB)], io, so[b]).wait()

    _build_cvec(i32(0), 0)
    _start_gathers(0)

    def outer(tt, carry):
        for b in range(2):
            u = tt * 2 + b
            nb = 1 - b
            _wait_gathers(b)

            @pl.when(u + 1 < _NUNIT)
            def _():
                _build_cvec(u + 1, nb)
                _start_gathers(nb)

            @pl.when(u >= 2)
            def _():
                _wait_outs(u - 2, b)

            _transpose(b)
            _issue_outs(u, b)
        return carry

    lax.fori_loop(jnp.int32(0), jnp.int32(_NUNIT // 2), outer, 0)

    for b in range(2):
        _wait_outs(_NUNIT - 2 + b, b)


def kernel(x, real_weight, imag_weight):
    idx = x.reshape(_N).astype(jnp.int32)
    rtl = jnp.reshape(real_weight[_NFULL * _VC:], (_VTAIL * _D,))
    itl = jnp.reshape(imag_weight[_NFULL * _VC:], (_VTAIL * _D,))
    rpk, ipk = _sc_repack(real_weight.T, imag_weight.T, rtl, itl)
    rtab = jnp.reshape(rpk, (_VOCAB, _D))
    itab = jnp.reshape(ipk, (_VOCAB, _D))
    rp, ip = _sc_embed(idx, rtab, itab)
    r = jnp.transpose(rp, (2, 0, 1))
    i = jnp.transpose(ip, (2, 0, 1))
    return lax.complex(r, i)
